# baseline (speedup 1.0000x reference)
import functools

import jax
import jax.numpy as jnp
from jax import lax
from jax.experimental import pallas as pl
from jax.experimental.pallas import tpu as pltpu
from jax.experimental.pallas import tpu_sc as plsc

_NC, _NS = 2, 16
_NW = _NC * _NS
_KB = 8
_NBUF = 2
_SPLIT = 128


def _embed_gather(idx, table):
    n_rows, seq = idx.shape
    depth = table.shape[1]
    rows_per_w = n_rows // _NW
    n_chunks = rows_per_w // _KB
    tail = seq - _SPLIT

    mesh = plsc.VectorSubcoreMesh(core_axis_name="c", subcore_axis_name="s")

    @functools.partial(
        pl.kernel,
        out_type=jax.ShapeDtypeStruct((n_rows, seq, depth), jnp.float32),
        mesh=mesh,
        scratch_types=[
            pltpu.VMEM((_NBUF, _KB, seq), jnp.int32),
            pltpu.VMEM((_NBUF, _KB, seq, depth), jnp.float32),
            pltpu.SemaphoreType.DMA((_NBUF,)),
            pltpu.SemaphoreType.DMA,
        ],
        compiler_params=pltpu.CompilerParams(use_tc_tiling_on_sc=False),
    )
    def run(idx_hbm, table_hbm, out_hbm, idx_v, rows_v, gsem, osem):
        wid = lax.axis_index("s") * _NC + lax.axis_index("c")
        row0 = wid * rows_per_w

        def gather_parts(b, j):
            return (
                (table_hbm.at[idx_v.at[b, j, pl.ds(0, _SPLIT)]],
                 rows_v.at[b, j, pl.ds(0, _SPLIT)]),
                (table_hbm.at[idx_v.at[b, j, pl.ds(_SPLIT, tail)]],
                 rows_v.at[b, j, pl.ds(_SPLIT, tail)]),
            )

        def stage_and_fire(c, b):
            r = row0 + c * _KB
            pltpu.sync_copy(idx_hbm.at[pl.ds(r, _KB)], idx_v.at[b])
            for j in range(_KB):
                for src, dst in gather_parts(b, j):
                    pltpu.async_copy(src, dst, gsem.at[b])

        def wait_gathers(b):
            for j in range(_KB):
                for src, dst in gather_parts(b, j):
                    pltpu.make_async_copy(src, dst, gsem.at[b]).wait()

        stage_and_fire(0, 0)
        stage_and_fire(1, 1)

        @pl.loop(0, n_chunks)
        def _chunk(c):
            b = lax.rem(c, _NBUF)
            r = row0 + c * _KB
            wait_gathers(b)
            pltpu.async_copy(rows_v.at[b], out_hbm.at[pl.ds(r, _KB)],
                             osem).wait()

            @pl.when(c + _NBUF < n_chunks)
            def _next():
                stage_and_fire(c + _NBUF, b)

    return run(idx, table)


def kernel(indices_tensor, table):
    return _embed_gather(indices_tensor.astype(jnp.int32), table)


# speedup vs baseline: 2.5320x; 2.5320x over previous
import functools

import jax
import jax.numpy as jnp
from jax import lax
from jax.experimental import pallas as pl
from jax.experimental.pallas import tpu as pltpu
from jax.experimental.pallas import tpu_sc as plsc

_NC, _NS = 2, 16
_NW = _NC * _NS
_BT = 128
_RB = 4
_OB = 4


def _embed_gather(idx_t, table):
    seq, n_rows = idx_t.shape
    depth = table.shape[1]
    ndt = depth // 8

    mesh = plsc.VectorSubcoreMesh(core_axis_name="c", subcore_axis_name="s")

    @functools.partial(
        pl.kernel,
        out_type=jax.ShapeDtypeStruct((seq, ndt, _NW, 8, _BT), jnp.float32),
        mesh=mesh,
        scratch_types=[
            pltpu.VMEM((seq, _BT), jnp.int32),
            pltpu.VMEM((_RB, _BT, depth), jnp.float32),
            pltpu.VMEM((_OB, ndt, 8, _BT), jnp.float32),
            pltpu.SemaphoreType.DMA((_RB,)),
            pltpu.SemaphoreType.DMA((_OB,)),
        ],
        compiler_params=pltpu.CompilerParams(use_tc_tiling_on_sc=False,
                                             needs_layout_passes=False),
    )
    def run(idx_hbm, table_hbm, out_hbm, idx_vt, rows_v, out_b, gsem, osem):
        wid = lax.axis_index("s") * _NC + lax.axis_index("c")
        col0 = wid * _BT

        pltpu.sync_copy(idx_hbm.at[:, pl.ds(col0, _BT)], idx_vt)

        def fire_gather(s, b):
            pltpu.async_copy(table_hbm.at[idx_vt.at[s]], rows_v.at[b],
                             gsem.at[b])

        def wait_gather(s, b):
            pltpu.make_async_copy(table_hbm.at[idx_vt.at[s]], rows_v.at[b],
                                  gsem.at[b]).wait()

        def fire_out(s, b):
            pltpu.async_copy(out_b.at[b], out_hbm.at[s, :, wid], osem.at[b])

        def wait_out(s, b):
            pltpu.make_async_copy(out_b.at[b], out_hbm.at[s, :, wid],
                                  osem.at[b]).wait()

        iota = lax.iota(jnp.int32, 16)

        for p in range(_RB):
            fire_gather(p, p)

        @pl.loop(0, seq)
        def _step(s):
            gb = lax.rem(s, _RB)
            ob = lax.rem(s, _OB)
            wait_gather(s, gb)

            @pl.when(s >= _OB)
            def _drain():
                wait_out(s - _OB, ob)

            rv = rows_v.at[gb]
            for d in range(depth):
                cols = jnp.full((16,), d, jnp.int32)
                for g in range(_BT // 16):
                    col = plsc.load_gather(rv, [g * 16 + iota, cols])
                    out_b[ob, d // 8, d % 8, pl.ds(g * 16, 16)] = col

            fire_out(s, ob)

            @pl.when(s + _RB < seq)
            def _refill():
                fire_gather(s + _RB, gb)

        @pl.loop(seq - _OB, seq)
        def _final(s):
            wait_out(s, lax.rem(s, _OB))

    return run(idx_t, table)


def kernel(indices_tensor, table):
    batch, seq = indices_tensor.shape
    depth = table.shape[1]
    idx_t = indices_tensor.astype(jnp.int32).T
    out5 = _embed_gather(idx_t, table)
    return out5.transpose(2, 4, 0, 1, 3).reshape(batch, seq, depth)


# speedup vs baseline: 3.2344x; 1.2774x over previous
import functools

import jax
import jax.numpy as jnp
from jax import lax
from jax.experimental import pallas as pl
from jax.experimental.pallas import tpu as pltpu
from jax.experimental.pallas import tpu_sc as plsc

_NC, _NS = 2, 16
_NW = _NC * _NS
_BT = 128
_OB = 4
_L = 16


def _embed_gather(idx_t, table):
    seq, n_rows = idx_t.shape
    vocab, depth = table.shape
    ndt = depth // 8

    mesh = plsc.VectorSubcoreMesh(core_axis_name="c", subcore_axis_name="s")

    @functools.partial(
        pl.kernel,
        out_type=jax.ShapeDtypeStruct((seq, ndt, _NW, 8, _BT), jnp.float32),
        mesh=mesh,
        scratch_types=[
            pltpu.VMEM((seq, _BT), jnp.int32),
            pltpu.VMEM((vocab, depth), jnp.float32),
            pltpu.VMEM((_OB, ndt, 8, _BT), jnp.float32),
            pltpu.SemaphoreType.DMA((_OB,)),
        ],
        compiler_params=pltpu.CompilerParams(use_tc_tiling_on_sc=False,
                                             needs_layout_passes=False),
    )
    def run(idx_hbm, table_hbm, out_hbm, idx_vt, table_v, out_b, osem):
        wid = lax.axis_index("s") * _NC + lax.axis_index("c")
        col0 = wid * _BT

        pltpu.sync_copy(table_hbm, table_v)
        pltpu.sync_copy(idx_hbm.at[:, pl.ds(col0, _BT)], idx_vt)

        def fire_out(s, b):
            pltpu.async_copy(out_b.at[b], out_hbm.at[s, :, wid], osem.at[b])

        def wait_out(s, b):
            pltpu.make_async_copy(out_b.at[b], out_hbm.at[s, :, wid],
                                  osem.at[b]).wait()

        @pl.loop(0, seq)
        def _step(s):
            ob = lax.rem(s, _OB)

            @pl.when(s >= _OB)
            def _drain():
                wait_out(s - _OB, ob)

            idxv = [idx_vt[s, pl.ds(g * _L, _L)] for g in range(_BT // _L)]
            for d in range(depth):
                cols = jnp.full((_L,), d, jnp.int32)
                vals = [plsc.load_gather(table_v, [idxv[g], cols])
                        for g in range(_BT // _L)]
                for g in range(_BT // _L):
                    out_b[ob, d // 8, d % 8, pl.ds(g * _L, _L)] = vals[g]

            fire_out(s, ob)

        @pl.loop(seq - _OB, seq)
        def _final(s):
            wait_out(s, lax.rem(s, _OB))

    return run(idx_t, table)


def kernel(indices_tensor, table):
    batch, seq = indices_tensor.shape
    depth = table.shape[1]
    idx_t = indices_tensor.astype(jnp.int32).T
    out5 = _embed_gather(idx_t, table)
    return out5.transpose(2, 4, 0, 1, 3).reshape(batch, seq, depth)


# speedup vs baseline: 8.9951x; 2.7811x over previous
import functools

import jax
import jax.numpy as jnp
from jax import lax
from jax.experimental import pallas as pl
from jax.experimental.pallas import tpu as pltpu
from jax.experimental.pallas import tpu_sc as plsc

_NC, _NS = 2, 16
_NW = _NC * _NS
_BT = 128
_OB = 4
_L = 16


def _embed_gather(idx_t, table_t):
    seq, n_rows = idx_t.shape
    depth, vocab = table_t.shape
    ndt = depth // 8

    mesh = plsc.VectorSubcoreMesh(core_axis_name="c", subcore_axis_name="s")

    @functools.partial(
        pl.kernel,
        out_type=jax.ShapeDtypeStruct((seq, ndt, _NW, 8, _BT), jnp.float32),
        mesh=mesh,
        scratch_types=[
            pltpu.VMEM((seq, _BT), jnp.int32),
            pltpu.VMEM((depth, vocab), jnp.float32),
            pltpu.VMEM((_OB, ndt, 8, _BT), jnp.float32),
            pltpu.SemaphoreType.DMA((_OB,)),
        ],
        compiler_params=pltpu.CompilerParams(use_tc_tiling_on_sc=False,
                                             needs_layout_passes=False),
    )
    def run(idx_hbm, table_hbm, out_hbm, idx_vt, table_v, out_b, osem):
        wid = lax.axis_index("s") * _NC + lax.axis_index("c")
        col0 = wid * _BT

        pltpu.sync_copy(table_hbm, table_v)
        pltpu.sync_copy(idx_hbm.at[:, pl.ds(col0, _BT)], idx_vt)

        def fire_out(s, b):
            pltpu.async_copy(out_b.at[b], out_hbm.at[s, :, wid], osem.at[b])

        def wait_out(s, b):
            pltpu.make_async_copy(out_b.at[b], out_hbm.at[s, :, wid],
                                  osem.at[b]).wait()

        @pl.loop(0, seq)
        def _step(s):
            ob = lax.rem(s, _OB)

            @pl.when(s >= _OB)
            def _drain():
                wait_out(s - _OB, ob)

            idxv = [idx_vt[s, pl.ds(g * _L, _L)] for g in range(_BT // _L)]
            for d in range(depth):
                cols = jnp.full((_L,), d, jnp.int32)
                vals = [plsc.load_gather(table_v, [cols, idxv[g]])
                        for g in range(_BT // _L)]
                for g in range(_BT // _L):
                    out_b[ob, d // 8, d % 8, pl.ds(g * _L, _L)] = vals[g]

            fire_out(s, ob)

        @pl.loop(seq - _OB, seq)
        def _final(s):
            wait_out(s, lax.rem(s, _OB))

    return run(idx_t, table_t)


def kernel(indices_tensor, table):
    batch, seq = indices_tensor.shape
    depth = table.shape[1]
    idx_t = indices_tensor.astype(jnp.int32).T
    out5 = _embed_gather(idx_t, table.T)
    return out5.transpose(2, 4, 0, 1, 3).reshape(batch, seq, depth)


# speedup vs baseline: 9.4641x; 1.0521x over previous
import functools

import jax
import jax.numpy as jnp
from jax import lax
from jax.experimental import pallas as pl
from jax.experimental.pallas import tpu as pltpu
from jax.experimental.pallas import tpu_sc as plsc

_NC, _NS = 2, 16
_NW = _NC * _NS
_BT = 128
_OB = 4
_L = 16


def _embed_gather(idx4, table_t):
    nst, nbt, sst, _ = idx4.shape
    seq = nst * sst
    depth, vocab = table_t.shape
    ndt = depth // 8

    mesh = plsc.VectorSubcoreMesh(core_axis_name="c", subcore_axis_name="s")

    @functools.partial(
        pl.kernel,
        out_type=jax.ShapeDtypeStruct((seq, ndt, _NW, 8, _BT), jnp.float32),
        mesh=mesh,
        scratch_types=[
            pltpu.VMEM((nst, sst, _BT), jnp.int32),
            pltpu.VMEM((depth, vocab), jnp.float32),
            pltpu.VMEM((_OB, ndt, 8, _BT), jnp.float32),
            pltpu.SemaphoreType.DMA((_OB,)),
        ],
        compiler_params=pltpu.CompilerParams(use_tc_tiling_on_sc=False,
                                             needs_layout_passes=False),
    )
    def run(idx_hbm, table_hbm, out_hbm, idx_vt, table_v, out_b, osem):
        wid = lax.axis_index("s") * _NC + lax.axis_index("c")

        pltpu.sync_copy(table_hbm, table_v)
        pltpu.sync_copy(idx_hbm.at[:, wid], idx_vt)

        def fire_out(s, b):
            pltpu.async_copy(out_b.at[b], out_hbm.at[s, :, wid], osem.at[b])

        def wait_out(s, b):
            pltpu.make_async_copy(out_b.at[b], out_hbm.at[s, :, wid],
                                  osem.at[b]).wait()

        @pl.loop(0, seq)
        def _step(s):
            ob = lax.rem(s, _OB)

            @pl.when(s >= _OB)
            def _drain():
                wait_out(s - _OB, ob)

            st = lax.div(s, sst)
            ss = lax.rem(s, sst)
            idxv = [idx_vt[st, ss, pl.ds(g * _L, _L)]
                    for g in range(_BT // _L)]
            for d in range(depth):
                cols = jnp.full((_L,), d, jnp.int32)
                vals = [plsc.load_gather(table_v, [cols, idxv[g]])
                        for g in range(_BT // _L)]
                for g in range(_BT // _L):
                    out_b[ob, d // 8, d % 8, pl.ds(g * _L, _L)] = vals[g]

            fire_out(s, ob)

        @pl.loop(seq - _OB, seq)
        def _final(s):
            wait_out(s, lax.rem(s, _OB))

    return run(idx4, table_t)


def kernel(indices_tensor, table):
    batch, seq = indices_tensor.shape
    depth = table.shape[1]
    idx4 = (indices_tensor.astype(jnp.int32).T
            .reshape(seq // 8, 8, batch // _BT, _BT)
            .transpose(0, 2, 1, 3))
    out5 = _embed_gather(idx4, table.T)
    return out5.transpose(2, 4, 0, 1, 3).reshape(batch, seq, depth)
